# baseline (speedup 1.0000x reference)
import functools
import math

import jax
import jax.numpy as jnp
from jax import lax
from jax.experimental import pallas as pl
from jax.experimental.pallas import tpu as pltpu
from jax.experimental.pallas import tpu_sc as plsc

_EMBED_DIM = 64
_SCALE = math.sqrt(_EMBED_DIM)
_NC = 2
_NS = 16
_LANES = 16
_NW = _NC * _NS
_W = 128


@functools.lru_cache(maxsize=None)
def _make_kernel(B: int):
    assert B % (_NW * _W) == 0
    b_per_w = B // _NW
    n_steps = b_per_w // _W
    mesh = plsc.VectorSubcoreMesh(core_axis_name="c", subcore_axis_name="s")

    @functools.partial(
        pl.kernel,
        mesh=mesh,
        compiler_params=pltpu.CompilerParams(use_tc_tiling_on_sc=False),
        out_type=jax.ShapeDtypeStruct((B, _EMBED_DIM), jnp.float32),
        scratch_types=[
            pltpu.VMEM((_W,), jnp.int32),
            pltpu.VMEM((_W, _EMBED_DIM), jnp.float32),
            pltpu.SemaphoreType.DMA,
        ],
    )
    def gather_scale(idx_hbm, table_hbm, out_hbm, idx_v, rows_v, sem):
        wid = lax.axis_index("s") * _NC + lax.axis_index("c")
        base = wid * b_per_w

        @pl.loop(0, n_steps)
        def _(step):
            off = base + step * _W
            pltpu.sync_copy(idx_hbm.at[pl.ds(off, _W)], idx_v)
            pltpu.async_copy(table_hbm.at[idx_v], rows_v, sem).wait()

            @pl.loop(0, _W)
            def _(r):
                for c in range(0, _EMBED_DIM, _LANES):
                    sl = (r, pl.ds(c, _LANES))
                    rows_v.at[sl][...] = rows_v.at[sl][...] * _SCALE

            pltpu.sync_copy(rows_v, out_hbm.at[pl.ds(off, _W)])

    return gather_scale


def kernel(input_vector, table):
    n, s = input_vector.shape
    B = n * s
    idx = input_vector.reshape(B).astype(jnp.int32)
    out = _make_kernel(B)(idx, table)
    return out.reshape(n, s, _EMBED_DIM)


# speedup vs baseline: 1.2765x; 1.2765x over previous
import functools
import math

import jax
import jax.numpy as jnp
from jax import lax
from jax.experimental import pallas as pl
from jax.experimental.pallas import tpu as pltpu
from jax.experimental.pallas import tpu_sc as plsc

_EMBED_DIM = 64
_SCALE = math.sqrt(_EMBED_DIM)
_NC = 2
_NS = 16
_LANES = 16
_NW = _NC * _NS
_W = 128
_NBUF = 4


@functools.lru_cache(maxsize=None)
def _make_kernel(B: int):
    assert B % (_NW * _W * _NBUF) == 0
    b_per_w = B // _NW
    n_steps = b_per_w // _W
    mesh = plsc.VectorSubcoreMesh(core_axis_name="c", subcore_axis_name="s")

    rows_t = pltpu.VMEM((_W, _EMBED_DIM), jnp.float32)

    @functools.partial(
        pl.kernel,
        mesh=mesh,
        compiler_params=pltpu.CompilerParams(use_tc_tiling_on_sc=False),
        out_type=jax.ShapeDtypeStruct((B, _EMBED_DIM), jnp.float32),
        scratch_types=(
            [pltpu.VMEM((b_per_w,), jnp.int32)]
            + [rows_t] * _NBUF
            + [rows_t] * _NBUF
            + [pltpu.SemaphoreType.DMA] * (2 * _NBUF)
        ),
    )
    def gather_scale(idx_hbm, table_hbm, out_hbm, idx_all, *bufs):
        gbuf = bufs[:_NBUF]
        wbuf = bufs[_NBUF:2 * _NBUF]
        gsem = bufs[2 * _NBUF:3 * _NBUF]
        wsem = bufs[3 * _NBUF:]

        wid = lax.axis_index("s") * _NC + lax.axis_index("c")
        base = wid * b_per_w

        pltpu.sync_copy(idx_hbm.at[pl.ds(base, b_per_w)], idx_all)

        def gather(s, b):
            return pltpu.make_async_copy(
                table_hbm.at[idx_all.at[pl.ds(s * _W, _W)]], gbuf[b], gsem[b]
            )

        def write(s, k):
            return pltpu.make_async_copy(
                wbuf[k], out_hbm.at[pl.ds(base + s * _W, _W)], wsem[k]
            )

        for b in range(_NBUF):
            gather(b, b).start()

        @pl.loop(0, n_steps, step=_NBUF)
        def _(j):
            for b in range(_NBUF):
                s = j + b
                gather(s, b).wait()

                @pl.when(s >= _NBUF)
                def _():
                    write(s - _NBUF, b).wait()

                @pl.loop(0, _W)
                def _(r):
                    for c in range(0, _EMBED_DIM, _LANES):
                        sl = (r, pl.ds(c, _LANES))
                        wbuf[b].at[sl][...] = gbuf[b].at[sl][...] * _SCALE

                write(s, b).start()

                @pl.when(s + _NBUF < n_steps)
                def _():
                    gather(s + _NBUF, b).start()

        for b in range(_NBUF):
            write(n_steps - _NBUF + b, b).wait()

    return gather_scale


def kernel(input_vector, table):
    n, s = input_vector.shape
    B = n * s
    idx = input_vector.reshape(B).astype(jnp.int32)
    out = _make_kernel(B)(idx, table)
    return out.reshape(n, s, _EMBED_DIM)
